# baseline (speedup 1.0000x reference)
import functools

import jax
import jax.numpy as jnp
from jax import lax
from jax.experimental import pallas as pl
from jax.experimental.pallas import tpu as pltpu
from jax.experimental.pallas import tpu_sc as plsc

_COL_BLOCK = 2048
_K_TC = 13 * _COL_BLOCK
_LANES = 16


def _colsum_body(train_ref, score_ref):
    ones = jnp.ones((train_ref.shape[1],), jnp.float32)
    score_ref[...] = jax.lax.dot_general(
        train_ref[...],
        ones,
        (((1,), (0,)), ((), ())),
        preferred_element_type=jnp.float32,
    )


def _tc_colsum_t(train_t):
    n_items, n_users = train_t.shape
    return pl.pallas_call(
        _colsum_body,
        grid=(pl.cdiv(n_items, _COL_BLOCK),),
        in_specs=[pl.BlockSpec((_COL_BLOCK, n_users), lambda j: (j, 0))],
        out_specs=pl.BlockSpec((_COL_BLOCK,), lambda j: (j,)),
        out_shape=jax.ShapeDtypeStruct((n_items,), jnp.float32),
    )(train_t)


@functools.cache
def _make_sc_colsum(n_rows, n_cols, col_off, n_cols_sc):
    info = plsc.get_sparse_core_info()
    nc = info.num_cores
    n_workers = nc * info.num_subcores
    w = -(-n_cols_sc // (n_workers * 128)) * 128
    n_full = n_cols_sc // w
    w_tail = n_cols_sc - n_full * w
    rchunk = 16
    rchunk_t = 8
    assert col_off % 128 == 0 and w % 128 == 0 and n_full <= n_workers
    assert (rchunk * w * 2 + w + rchunk_t * max(w_tail, 1) * 2 + max(w_tail, 1)) * 4 < 500_000
    mesh = plsc.VectorSubcoreMesh(core_axis_name="c", subcore_axis_name="s")

    @functools.partial(
        pl.kernel,
        mesh=mesh,
        out_type=jax.ShapeDtypeStruct((n_cols_sc,), jnp.float32),
        scratch_types=[
            pltpu.VMEM((rchunk, w), jnp.float32),
            pltpu.VMEM((rchunk, w), jnp.float32),
            pltpu.VMEM((w,), jnp.float32),
            pltpu.VMEM((rchunk_t, max(w_tail, _LANES)), jnp.float32),
            pltpu.VMEM((rchunk_t, max(w_tail, _LANES)), jnp.float32),
            pltpu.VMEM((max(w_tail, _LANES),), jnp.float32),
            pltpu.SemaphoreType.DMA,
            pltpu.SemaphoreType.DMA,
        ],
    )
    def colsum_kernel(
        train_hbm, score_hbm, buf0, buf1, acc_v, tbuf0, tbuf1, tacc_v, sem0, sem1
    ):
        wid = lax.axis_index("s") * nc + lax.axis_index("c")
        sems = (sem0, sem1)

        def run_stripe(bufs, acc, rows, crel, w_eff):
            nv_eff = w_eff // _LANES
            nchunk = n_rows // rows

            def chunk_src(g):
                return train_hbm.at[
                    pl.ds(g * rows, rows), pl.ds(col_off + crel, w_eff)
                ]

            def start(g, b):
                pltpu.async_copy(chunk_src(g), bufs[b], sems[b])

            def wait(g, b):
                pltpu.make_async_copy(chunk_src(g), bufs[b], sems[b]).wait()

            @plsc.parallel_loop(0, nv_eff, unroll=4)
            def _(v):
                acc[pl.ds(v * _LANES, _LANES)] = jnp.zeros((_LANES,), jnp.float32)

            start(0, 0)
            start(1, 1)

            def accumulate(buf):
                @plsc.parallel_loop(0, nv_eff, unroll=4)
                def _(v):
                    s = v * _LANES
                    vs = [buf[r, pl.ds(s, _LANES)] for r in range(rows)]
                    while len(vs) > 1:
                        vs = [vs[i] + vs[i + 1] for i in range(0, len(vs), 2)]
                    acc[pl.ds(s, _LANES)] = acc[pl.ds(s, _LANES)] + vs[0]

            def body(p, carry):
                g = 2 * p
                wait(g, 0)
                accumulate(bufs[0])

                @pl.when(g + 2 < nchunk)
                def _():
                    start(g + 2, 0)

                wait(g + 1, 1)
                accumulate(bufs[1])

                @pl.when(g + 3 < nchunk)
                def _():
                    start(g + 3, 1)

                return carry

            lax.fori_loop(0, nchunk // 2, body, 0)
            pltpu.sync_copy(acc, score_hbm.at[pl.ds(crel, w_eff)])

        @pl.when(wid < n_full)
        def _():
            run_stripe((buf0, buf1), acc_v, rchunk, wid * w, w)

        if w_tail:

            @pl.when(wid == n_full)
            def _():
                run_stripe((tbuf0, tbuf1), tacc_v, rchunk_t, n_full * w, w_tail)

    return colsum_kernel


@functools.cache
def _make_gather(n_tc, n_sc, n_idx):
    info = plsc.get_sparse_core_info()
    n_workers = info.num_cores * info.num_subcores
    n_items = n_tc + n_sc
    per_w = n_idx // n_workers
    assert per_w * n_workers == n_idx and per_w % _LANES == 0
    mesh = plsc.VectorSubcoreMesh(core_axis_name="c", subcore_axis_name="s")

    @functools.partial(
        pl.kernel,
        mesh=mesh,
        out_type=jax.ShapeDtypeStruct((n_idx,), jnp.float32),
        scratch_types=[
            pltpu.VMEM((n_items,), jnp.float32),
            pltpu.VMEM((per_w,), jnp.int32),
            pltpu.VMEM((per_w,), jnp.float32),
        ],
        compiler_params=pltpu.CompilerParams(needs_layout_passes=False),
    )
    def gather_kernel(*refs):
        if n_sc:
            sa_hbm, sb_hbm, idx_hbm, out_hbm, table_v, idx_v, out_v = refs
        else:
            sa_hbm, idx_hbm, out_hbm, table_v, idx_v, out_v = refs
        wid = lax.axis_index("s") * info.num_cores + lax.axis_index("c")
        base = wid * per_w
        if n_sc:
            pltpu.sync_copy(sa_hbm, table_v.at[pl.ds(0, n_tc)])
            pltpu.sync_copy(sb_hbm, table_v.at[pl.ds(n_tc, n_sc)])
        else:
            pltpu.sync_copy(sa_hbm, table_v)
        pltpu.sync_copy(idx_hbm.at[pl.ds(base, per_w)], idx_v)

        def body(i, carry):
            iv = idx_v[pl.ds(i * _LANES, _LANES)]
            out_v[pl.ds(i * _LANES, _LANES)] = plsc.load_gather(table_v, [iv])
            return carry

        lax.fori_loop(0, per_w // _LANES, body, 0)
        pltpu.sync_copy(out_v, out_hbm.at[pl.ds(base, per_w)])

    return gather_kernel


def kernel(train, test_items):
    n_rows, n_cols = train.shape
    score = _tc_colsum_t(train.T)
    idx = test_items.reshape(-1).astype(jnp.int32)
    out = _make_gather(n_cols, 0, idx.shape[0])(score, idx)
    return out.reshape(test_items.shape)


# speedup vs baseline: 1.0254x; 1.0254x over previous
import functools

import jax
import jax.numpy as jnp
from jax import lax
from jax.experimental import pallas as pl
from jax.experimental.pallas import tpu as pltpu
from jax.experimental.pallas import tpu_sc as plsc

_COL_BLOCK = 2048
_K_TC = 13 * _COL_BLOCK
_LANES = 16


def _colsum_body(train_ref, score_ref):
    score_ref[...] = jnp.sum(train_ref[...], axis=1)


def _tc_colsum_t(train_t):
    n_items, n_users = train_t.shape
    return pl.pallas_call(
        _colsum_body,
        grid=(pl.cdiv(n_items, _COL_BLOCK),),
        in_specs=[pl.BlockSpec((_COL_BLOCK, n_users), lambda j: (j, 0))],
        out_specs=pl.BlockSpec((_COL_BLOCK,), lambda j: (j,)),
        out_shape=jax.ShapeDtypeStruct((n_items,), jnp.float32),
    )(train_t)


@functools.cache
def _make_sc_colsum(n_rows, n_cols, col_off, n_cols_sc):
    info = plsc.get_sparse_core_info()
    nc = info.num_cores
    n_workers = nc * info.num_subcores
    w = -(-n_cols_sc // (n_workers * 128)) * 128
    n_full = n_cols_sc // w
    w_tail = n_cols_sc - n_full * w
    rchunk = 16
    rchunk_t = 8
    assert col_off % 128 == 0 and w % 128 == 0 and n_full <= n_workers
    assert (rchunk * w * 2 + w + rchunk_t * max(w_tail, 1) * 2 + max(w_tail, 1)) * 4 < 500_000
    mesh = plsc.VectorSubcoreMesh(core_axis_name="c", subcore_axis_name="s")

    @functools.partial(
        pl.kernel,
        mesh=mesh,
        out_type=jax.ShapeDtypeStruct((n_cols_sc,), jnp.float32),
        scratch_types=[
            pltpu.VMEM((rchunk, w), jnp.float32),
            pltpu.VMEM((rchunk, w), jnp.float32),
            pltpu.VMEM((w,), jnp.float32),
            pltpu.VMEM((rchunk_t, max(w_tail, _LANES)), jnp.float32),
            pltpu.VMEM((rchunk_t, max(w_tail, _LANES)), jnp.float32),
            pltpu.VMEM((max(w_tail, _LANES),), jnp.float32),
            pltpu.SemaphoreType.DMA,
            pltpu.SemaphoreType.DMA,
        ],
    )
    def colsum_kernel(
        train_hbm, score_hbm, buf0, buf1, acc_v, tbuf0, tbuf1, tacc_v, sem0, sem1
    ):
        wid = lax.axis_index("s") * nc + lax.axis_index("c")
        sems = (sem0, sem1)

        def run_stripe(bufs, acc, rows, crel, w_eff):
            nv_eff = w_eff // _LANES
            nchunk = n_rows // rows

            def chunk_src(g):
                return train_hbm.at[
                    pl.ds(g * rows, rows), pl.ds(col_off + crel, w_eff)
                ]

            def start(g, b):
                pltpu.async_copy(chunk_src(g), bufs[b], sems[b])

            def wait(g, b):
                pltpu.make_async_copy(chunk_src(g), bufs[b], sems[b]).wait()

            @plsc.parallel_loop(0, nv_eff, unroll=4)
            def _(v):
                acc[pl.ds(v * _LANES, _LANES)] = jnp.zeros((_LANES,), jnp.float32)

            start(0, 0)
            start(1, 1)

            def accumulate(buf):
                @plsc.parallel_loop(0, nv_eff, unroll=4)
                def _(v):
                    s = v * _LANES
                    vs = [buf[r, pl.ds(s, _LANES)] for r in range(rows)]
                    while len(vs) > 1:
                        vs = [vs[i] + vs[i + 1] for i in range(0, len(vs), 2)]
                    acc[pl.ds(s, _LANES)] = acc[pl.ds(s, _LANES)] + vs[0]

            def body(p, carry):
                g = 2 * p
                wait(g, 0)
                accumulate(bufs[0])

                @pl.when(g + 2 < nchunk)
                def _():
                    start(g + 2, 0)

                wait(g + 1, 1)
                accumulate(bufs[1])

                @pl.when(g + 3 < nchunk)
                def _():
                    start(g + 3, 1)

                return carry

            lax.fori_loop(0, nchunk // 2, body, 0)
            pltpu.sync_copy(acc, score_hbm.at[pl.ds(crel, w_eff)])

        @pl.when(wid < n_full)
        def _():
            run_stripe((buf0, buf1), acc_v, rchunk, wid * w, w)

        if w_tail:

            @pl.when(wid == n_full)
            def _():
                run_stripe((tbuf0, tbuf1), tacc_v, rchunk_t, n_full * w, w_tail)

    return colsum_kernel


@functools.cache
def _make_gather(n_tc, n_sc, n_idx):
    info = plsc.get_sparse_core_info()
    n_workers = info.num_cores * info.num_subcores
    n_items = n_tc + n_sc
    per_w = n_idx // n_workers
    assert per_w * n_workers == n_idx and per_w % _LANES == 0
    mesh = plsc.VectorSubcoreMesh(core_axis_name="c", subcore_axis_name="s")

    @functools.partial(
        pl.kernel,
        mesh=mesh,
        out_type=jax.ShapeDtypeStruct((n_idx,), jnp.float32),
        scratch_types=[
            pltpu.VMEM((n_items,), jnp.float32),
            pltpu.VMEM((per_w,), jnp.int32),
            pltpu.VMEM((per_w,), jnp.float32),
        ],
        compiler_params=pltpu.CompilerParams(needs_layout_passes=False),
    )
    def gather_kernel(*refs):
        if n_sc:
            sa_hbm, sb_hbm, idx_hbm, out_hbm, table_v, idx_v, out_v = refs
        else:
            sa_hbm, idx_hbm, out_hbm, table_v, idx_v, out_v = refs
        wid = lax.axis_index("s") * info.num_cores + lax.axis_index("c")
        base = wid * per_w
        if n_sc:
            pltpu.sync_copy(sa_hbm, table_v.at[pl.ds(0, n_tc)])
            pltpu.sync_copy(sb_hbm, table_v.at[pl.ds(n_tc, n_sc)])
        else:
            pltpu.sync_copy(sa_hbm, table_v)
        pltpu.sync_copy(idx_hbm.at[pl.ds(base, per_w)], idx_v)

        def body(i, carry):
            iv = idx_v[pl.ds(i * _LANES, _LANES)]
            out_v[pl.ds(i * _LANES, _LANES)] = plsc.load_gather(table_v, [iv])
            return carry

        lax.fori_loop(0, per_w // _LANES, body, 0)
        pltpu.sync_copy(out_v, out_hbm.at[pl.ds(base, per_w)])

    return gather_kernel


def kernel(train, test_items):
    n_rows, n_cols = train.shape
    n_users, n_test = test_items.shape
    score = _tc_colsum_t(train.T)
    idx = test_items.T.reshape(-1).astype(jnp.int32)
    out = _make_gather(n_cols, 0, idx.shape[0])(score, idx)
    return out.reshape(n_test, n_users).T
